# baseline (speedup 1.0000x reference)
import functools

import jax
import jax.numpy as jnp
from jax import lax
from jax.experimental import pallas as pl
from jax.experimental.pallas import tpu as pltpu
from jax.experimental.pallas import tpu_sc as plsc

TOPK = 5
BATCH_N = 128
VOCAB_N = 100000
LANES = 16
NCORES = 2
NSUB = 16
NWORKERS = NCORES * NSUB
ROWS_PER = BATCH_N // NWORKERS
LAB_PAD = 32
BIGPOS = jnp.int32(1 << 30)

GU = 10
GW = GU * LANES
NG = VOCAB_N // GW
SUP = 5
NSUPER = NG // SUP
CHUNK_GROUPS = 125
CW = CHUNK_GROUPS * GW
NCHUNK = NG // CHUNK_GROUPS


def _insert5(t, v):
    t0, t1, t2, t3, t4 = t
    n0 = jnp.maximum(t0, v)
    r = jnp.minimum(t0, v)
    n1 = jnp.maximum(t1, r)
    r = jnp.minimum(t1, r)
    n2 = jnp.maximum(t2, r)
    r = jnp.minimum(t2, r)
    n3 = jnp.maximum(t3, r)
    r = jnp.minimum(t3, r)
    n4 = jnp.maximum(t4, r)
    return (n0, n1, n2, n3, n4)


def _tree_max(vs):
    vs = list(vs)
    while len(vs) > 1:
        nxt = [jnp.maximum(vs[i], vs[i + 1]) for i in range(0, len(vs) - 1, 2)]
        if len(vs) % 2:
            nxt.append(vs[-1])
        vs = nxt
    return vs[0]


def _merge_t5(merge_v, tops):
    for j in range(TOPK):
        merge_v[j] = tops[j]
    iota = lax.iota(jnp.int32, LANES)

    def sel_iter(_, t5_prev):
        mv = jnp.full((LANES,), -jnp.inf, jnp.float32)
        mp = jnp.full((LANES,), BIGPOS, jnp.int32)
        for j in range(TOPK):
            cur = merge_v[j]
            p = iota + jnp.int32(j * LANES)
            upd = (cur > mv) | ((cur == mv) & (p < mp))
            mv = jnp.where(upd, cur, mv)
            mp = jnp.where(upd, p, mp)
        mvs = jnp.max(mv)
        mps = jnp.min(jnp.where(mv == mvs, mp, BIGPOS))
        for j in range(TOPK):
            cur = merge_v[j]
            p = iota + jnp.int32(j * LANES)
            merge_v[j] = jnp.where(p == mps, -jnp.inf, cur)
        return mvs

    return lax.fori_loop(0, TOPK, sel_iter, jnp.float32(0.0))


@functools.partial(
    pl.kernel,
    out_type=jax.ShapeDtypeStruct((NCORES, LANES), jnp.float32),
    mesh=plsc.VectorSubcoreMesh(core_axis_name="c", subcore_axis_name="s"),
    compiler_params=pltpu.CompilerParams(
        needs_layout_passes=False, use_tc_tiling_on_sc=False),
    scratch_types=[
        pltpu.VMEM((VOCAB_N,), jnp.float32),
        pltpu.VMEM((NG * LANES,), jnp.float32),
        pltpu.VMEM((LAB_PAD,), jnp.int32),
        pltpu.VMEM((TOPK, LANES), jnp.float32),
        pltpu.VMEM((LANES,), jnp.float32),
        pltpu.VMEM((NSUB, LANES), jnp.float32),
        pltpu.VMEM((LANES,), jnp.float32),
        pltpu.VMEM_SHARED((NSUB, LANES), jnp.float32),
        pltpu.SemaphoreType.DMA,
        pltpu.SemaphoreType.DMA,
    ],
)
def _sc_topk_acc(probas_hbm, labels_hbm, out_hbm,
                 row_v, gm_v, lab_v, merge_v, cnt_v, sums_v, out_v, shared,
                 sem0, sem1):
    core = lax.axis_index("c")
    sid = lax.axis_index("s")
    wid = sid * NCORES + core
    sems = (sem0, sem1)
    neg = jnp.full((LANES,), -jnp.inf, jnp.float32)

    def row_body(i, count):
        r = wid * ROWS_PER + i
        pltpu.sync_copy(labels_hbm.at[r], lab_v)

        def chunk_copy(c):
            return pltpu.make_async_copy(
                probas_hbm.at[r, pl.ds(c * CW, CW)],
                row_v.at[pl.ds(c * CW, CW)],
                sems[c % 2],
            )

        chunk_copy(0).start()
        chunk_copy(1).start()

        def p1(g, carry):
            base = g * GW
            vs = [row_v[pl.ds(base + j * LANES, LANES)] for j in range(GU)]
            vm = _tree_max(vs)
            gm_v[pl.ds(g * LANES, LANES)] = vm
            return _insert5(carry, vm)

        carry = (neg, neg, neg, neg, neg)
        for c in range(NCHUNK):
            chunk_copy(c).wait()
            if c + 2 < NCHUNK:
                chunk_copy(c + 2).start()
            carry = lax.fori_loop(
                c * CHUNK_GROUPS, (c + 1) * CHUNK_GROUPS, p1, carry)

        tau5 = _merge_t5(merge_v, carry)

        def insert_group(g, c):
            base = g * GW
            for j in range(GU):
                c = _insert5(c, row_v[pl.ds(base + j * LANES, LANES)])
            return c

        def p2(s, carry):
            base = s * (SUP * LANES)
            ms = [gm_v[pl.ds(base + j * LANES, LANES)] for j in range(SUP)]
            mm = _tree_max(ms)

            def hit(c):
                for j in range(SUP):
                    pj = jnp.any(ms[j] >= tau5)
                    c = lax.cond(
                        pj,
                        functools.partial(insert_group, s * SUP + j),
                        lambda cc: cc,
                        c,
                    )
                return c

            return lax.cond(jnp.any(mm >= tau5), hit, lambda c: c, carry)

        carry2 = lax.fori_loop(0, NSUPER, p2, (neg, neg, neg, neg, neg))
        t5 = _merge_t5(merge_v, carry2)

        g0 = plsc.load_gather(row_v, [lab_v[pl.ds(0, LANES)]])
        g1 = plsc.load_gather(row_v, [lab_v[pl.ds(LANES, LANES)]])
        lmax = jnp.max(jnp.maximum(g0, g1))
        return count + jnp.where(lmax >= t5, jnp.float32(1.0), jnp.float32(0.0))

    count = lax.fori_loop(0, ROWS_PER, row_body, jnp.float32(0.0))

    cnt_v[...] = jnp.broadcast_to(count, (LANES,))
    pltpu.sync_copy(cnt_v, shared.at[sid])
    plsc.subcore_barrier()

    @pl.when(sid == 0)
    def _():
        pltpu.sync_copy(shared, sums_v)
        tot = sums_v[0]
        for j in range(1, NSUB):
            tot = tot + sums_v[j]
        out_v[...] = tot
        pltpu.sync_copy(out_v, out_hbm.at[core])


def kernel(probas, labels):
    lab32 = jnp.concatenate(
        [labels, jnp.broadcast_to(labels[:, :1], (BATCH_N, LAB_PAD - labels.shape[1]))],
        axis=1,
    )
    out = _sc_topk_acc(probas, lab32)
    return (out[0, 0] + out[1, 0]) * jnp.float32(1.0 / BATCH_N)


# speedup vs baseline: 1.4155x; 1.4155x over previous
import functools

import jax
import jax.numpy as jnp
from jax import lax
from jax.experimental import pallas as pl
from jax.experimental.pallas import tpu as pltpu
from jax.experimental.pallas import tpu_sc as plsc

TOPK = 5
BATCH_N = 128
VOCAB_N = 100000
LANES = 16
NCORES = 2
NSUB = 16
NWORKERS = NCORES * NSUB
ROWS_PER = BATCH_N // NWORKERS
LAB_PAD = 32
BIGPOS = jnp.int32(1 << 30)

GU = 10
GW = GU * LANES
NG = VOCAB_N // GW
SUP = 5
NSUPER = NG // SUP
CHUNK_GROUPS = 125
CW = CHUNK_GROUPS * GW
NCHUNK = NG // CHUNK_GROUPS


def _insert5(t, v):
    t0, t1, t2, t3, t4 = t
    n0 = jnp.maximum(t0, v)
    r = jnp.minimum(t0, v)
    n1 = jnp.maximum(t1, r)
    r = jnp.minimum(t1, r)
    n2 = jnp.maximum(t2, r)
    r = jnp.minimum(t2, r)
    n3 = jnp.maximum(t3, r)
    r = jnp.minimum(t3, r)
    n4 = jnp.maximum(t4, r)
    return (n0, n1, n2, n3, n4)


def _tree_max(vs):
    vs = list(vs)
    while len(vs) > 1:
        nxt = [jnp.maximum(vs[i], vs[i + 1]) for i in range(0, len(vs) - 1, 2)]
        if len(vs) % 2:
            nxt.append(vs[-1])
        vs = nxt
    return vs[0]


def _merge_t5(merge_v, tops):
    for j in range(TOPK):
        merge_v[j] = tops[j]
    iota = lax.iota(jnp.int32, LANES)

    def sel_iter(_, t5_prev):
        mv = jnp.full((LANES,), -jnp.inf, jnp.float32)
        mp = jnp.full((LANES,), BIGPOS, jnp.int32)
        for j in range(TOPK):
            cur = merge_v[j]
            p = iota + jnp.int32(j * LANES)
            upd = (cur > mv) | ((cur == mv) & (p < mp))
            mv = jnp.where(upd, cur, mv)
            mp = jnp.where(upd, p, mp)
        mvs = jnp.max(mv)
        mps = jnp.min(jnp.where(mv == mvs, mp, BIGPOS))
        for j in range(TOPK):
            cur = merge_v[j]
            p = iota + jnp.int32(j * LANES)
            merge_v[j] = jnp.where(p == mps, -jnp.inf, cur)
        return mvs

    return lax.fori_loop(0, TOPK, sel_iter, jnp.float32(0.0))


@functools.partial(
    pl.kernel,
    out_type=jax.ShapeDtypeStruct((NCORES, LANES), jnp.float32),
    mesh=plsc.VectorSubcoreMesh(core_axis_name="c", subcore_axis_name="s"),
    compiler_params=pltpu.CompilerParams(needs_layout_passes=False),
    scratch_types=[
        pltpu.VMEM((VOCAB_N,), jnp.float32),
        pltpu.VMEM((NG * LANES,), jnp.float32),
        pltpu.VMEM((LAB_PAD,), jnp.int32),
        pltpu.VMEM((TOPK, LANES), jnp.float32),
        pltpu.VMEM((LANES,), jnp.float32),
        pltpu.VMEM((NSUB, LANES), jnp.float32),
        pltpu.VMEM((LANES,), jnp.float32),
        pltpu.VMEM_SHARED((NSUB, LANES), jnp.float32),
        pltpu.SemaphoreType.DMA,
        pltpu.SemaphoreType.DMA,
    ],
)
def _sc_topk_acc(probas_hbm, labels_hbm, out_hbm,
                 row_v, gm_v, lab_v, merge_v, cnt_v, sums_v, out_v, shared,
                 sem0, sem1):
    core = lax.axis_index("c")
    sid = lax.axis_index("s")
    wid = sid * NCORES + core
    sems = (sem0, sem1)
    neg = jnp.full((LANES,), -jnp.inf, jnp.float32)

    def row_body(i, count):
        r = wid * ROWS_PER + i
        pltpu.sync_copy(labels_hbm.at[r], lab_v)

        pltpu.sync_copy(probas_hbm.at[r], row_v)

        def p1(g, carry):
            base = g * GW
            vs = [row_v[pl.ds(base + j * LANES, LANES)] for j in range(GU)]
            vm = _tree_max(vs)
            gm_v[pl.ds(g * LANES, LANES)] = vm
            return _insert5(carry, vm)

        carry = lax.fori_loop(0, NG, p1, (neg, neg, neg, neg, neg))

        tau5 = _merge_t5(merge_v, carry)

        def insert_group(g, c):
            base = g * GW
            for j in range(GU):
                c = _insert5(c, row_v[pl.ds(base + j * LANES, LANES)])
            return c

        def p2(s, carry):
            base = s * (SUP * LANES)
            ms = [gm_v[pl.ds(base + j * LANES, LANES)] for j in range(SUP)]
            mm = _tree_max(ms)

            def hit(c):
                for j in range(SUP):
                    pj = jnp.any(ms[j] >= tau5)
                    c = lax.cond(
                        pj,
                        functools.partial(insert_group, s * SUP + j),
                        lambda cc: cc,
                        c,
                    )
                return c

            return lax.cond(jnp.any(mm >= tau5), hit, lambda c: c, carry)

        carry2 = lax.fori_loop(0, NSUPER, p2, (neg, neg, neg, neg, neg))
        t5 = _merge_t5(merge_v, carry2)

        g0 = plsc.load_gather(row_v, [lab_v[pl.ds(0, LANES)]])
        g1 = plsc.load_gather(row_v, [lab_v[pl.ds(LANES, LANES)]])
        lmax = jnp.max(jnp.maximum(g0, g1))
        return count + jnp.where(lmax >= t5, jnp.float32(1.0), jnp.float32(0.0))

    count = lax.fori_loop(0, ROWS_PER, row_body, jnp.float32(0.0))

    cnt_v[...] = jnp.broadcast_to(count, (LANES,))
    pltpu.sync_copy(cnt_v, shared.at[sid])
    plsc.subcore_barrier()

    @pl.when(sid == 0)
    def _():
        pltpu.sync_copy(shared, sums_v)
        tot = sums_v[0]
        for j in range(1, NSUB):
            tot = tot + sums_v[j]
        out_v[...] = tot
        pltpu.sync_copy(out_v, out_hbm.at[core])


def kernel(probas, labels):
    lab32 = jnp.concatenate(
        [labels, jnp.broadcast_to(labels[:, :1], (BATCH_N, LAB_PAD - labels.shape[1]))],
        axis=1,
    )
    out = _sc_topk_acc(probas, lab32)
    return (out[0, 0] + out[1, 0]) * jnp.float32(1.0 / BATCH_N)


# speedup vs baseline: 3.4324x; 2.4249x over previous
import functools

import jax
import jax.numpy as jnp
from jax import lax
from jax.experimental import pallas as pl
from jax.experimental.pallas import tpu as pltpu
from jax.experimental.pallas import tpu_sc as plsc

TOPK = 5
BATCH_N = 128
VOCAB_N = 100000
LANES = 16
NCORES = 2
NSUB = 16
NWORKERS = NCORES * NSUB
NLAB = 20
NBG = BATCH_N // LANES
SLAB = VOCAB_N // NWORKERS
CH_ROWS = 125
NCHUNK = SLAB // CH_ROWS
CH_W = CH_ROWS * BATCH_N
UNROLL = 5
INNER = CH_ROWS // UNROLL

_params = pltpu.CompilerParams(
    needs_layout_passes=False, use_tc_tiling_on_sc=False)
_mesh = plsc.VectorSubcoreMesh(core_axis_name="c", subcore_axis_name="s")


@functools.partial(
    pl.kernel,
    out_type=jax.ShapeDtypeStruct((NWORKERS, BATCH_N), jnp.int32),
    mesh=_mesh,
    compiler_params=_params,
    scratch_types=[
        pltpu.VMEM((2, CH_W), jnp.float32),
        pltpu.VMEM((NLAB, BATCH_N), jnp.int32),
        pltpu.VMEM((NLAB, BATCH_N), jnp.int32),
        pltpu.VMEM((NLAB, BATCH_N), jnp.float32),
        pltpu.VMEM((NBG, LANES), jnp.float32),
        pltpu.VMEM((NBG, UNROLL * LANES), jnp.int32),
        pltpu.VMEM((BATCH_N,), jnp.int32),
        pltpu.SemaphoreType.DMA,
        pltpu.SemaphoreType.DMA,
        pltpu.SemaphoreType.DMA,
    ],
)
def _sc_count(pflat_hbm, labt_hbm, cnt_hbm,
              buf, labv, idxv, lval, lmax_v, acc_v, out_v,
              sem0, sem1, gsem):
    core = lax.axis_index("c")
    sid = lax.axis_index("s")
    wid = sid * NCORES + core
    sems = (sem0, sem1)
    iota = lax.iota(jnp.int32, LANES)

    slab_base = wid * (SLAB * BATCH_N)

    def chunk_cp(c):
        return pltpu.make_async_copy(
            pflat_hbm.at[pl.ds(slab_base + c * CH_W, CH_W)],
            buf.at[c & 1],
            sems[c & 1],
        )

    chunk_cp(0).start()
    chunk_cp(1).start()

    pltpu.sync_copy(labt_hbm, labv)
    for j in range(NLAB):
        for bg in range(NBG):
            lab = labv[j, pl.ds(bg * LANES, LANES)]
            idx = lab * BATCH_N + (bg * LANES + iota)
            idxv[j, pl.ds(bg * LANES, LANES)] = idx
    gathers = [
        pltpu.make_async_copy(pflat_hbm.at[idxv.at[j]], lval.at[j], gsem)
        for j in range(NLAB)
    ]
    for g in gathers:
        g.start()
    for g in gathers:
        g.wait()
    for bg in range(NBG):
        m = lval[0, pl.ds(bg * LANES, LANES)]
        for j in range(1, NLAB):
            m = jnp.maximum(m, lval[j, pl.ds(bg * LANES, LANES)])
        lmax_v[bg] = m

    zero = jnp.zeros((LANES,), jnp.int32)
    for bg in range(NBG):
        for k in range(UNROLL):
            acc_v[bg, pl.ds(k * LANES, LANES)] = zero

    ones = jnp.ones((LANES,), jnp.int32)
    zeros = jnp.zeros((LANES,), jnp.int32)

    for c in range(NCHUNK):
        chunk_cp(c).wait()
        if c + 2 < NCHUNK:
            chunk_cp(c + 2).start()
        cb = c & 1

        def bg_body(bg, _, cb=cb):
            lmax = lmax_v[bg]
            accs = [acc_v[bg, pl.ds(k * LANES, LANES)] for k in range(UNROLL)]

            def row_body(i, accs, cb=cb, bg=bg, lmax=lmax):
                base = i * (UNROLL * BATCH_N) + bg * LANES
                out = []
                for k in range(UNROLL):
                    v = buf[cb, pl.ds(base + k * BATCH_N, LANES)]
                    out.append(accs[k] + jnp.where(v > lmax, ones, zeros))
                return tuple(out)

            accs = lax.fori_loop(0, INNER, row_body, tuple(accs))
            for k in range(UNROLL):
                acc_v[bg, pl.ds(k * LANES, LANES)] = accs[k]
            return 0

        lax.fori_loop(0, NBG, bg_body, 0)

    for bg in range(NBG):
        tot = acc_v[bg, pl.ds(0, LANES)]
        for k in range(1, UNROLL):
            tot = tot + acc_v[bg, pl.ds(k * LANES, LANES)]
        out_v[pl.ds(bg * LANES, LANES)] = tot
    pltpu.sync_copy(out_v, cnt_hbm.at[wid])


@functools.partial(
    pl.kernel,
    out_type=jax.ShapeDtypeStruct((LANES,), jnp.float32),
    mesh=_mesh,
    compiler_params=_params,
    scratch_types=[
        pltpu.VMEM((NWORKERS, BATCH_N), jnp.int32),
        pltpu.VMEM((LANES,), jnp.float32),
        pltpu.VMEM((NSUB, LANES), jnp.float32),
        pltpu.VMEM((LANES,), jnp.float32),
        pltpu.VMEM_SHARED((NSUB, LANES), jnp.float32),
    ],
)
def _sc_merge(cnt_hbm, out_hbm, cbuf, cnt_v, sums_v, outv, shared):
    core = lax.axis_index("c")
    sid = lax.axis_index("s")
    active = (core == 0) & (sid < NBG)

    @pl.when(active)
    def _():
        pltpu.sync_copy(cnt_hbm, cbuf)

    bg_off = jnp.minimum(sid, NBG - 1) * LANES
    tot = cbuf[0, pl.ds(bg_off, LANES)]
    for w in range(1, NWORKERS):
        tot = tot + cbuf[w, pl.ds(bg_off, LANES)]
    matches = jnp.sum(
        jnp.where(tot < TOPK, jnp.float32(1.0), jnp.float32(0.0)))
    nmatch = jnp.where(active, matches, jnp.float32(0.0))

    cnt_v[...] = jnp.broadcast_to(nmatch, (LANES,))
    pltpu.sync_copy(cnt_v, shared.at[sid])
    plsc.subcore_barrier()

    @pl.when((core == 0) & (sid == 0))
    def _():
        pltpu.sync_copy(shared, sums_v)
        tot = sums_v[0]
        for j in range(1, NSUB):
            tot = tot + sums_v[j]
        outv[...] = tot
        pltpu.sync_copy(outv, out_hbm)


def kernel(probas, labels):
    pflat = probas.T.reshape(-1)
    labt = labels.T
    cnt = _sc_count(pflat, labt)
    tot = _sc_merge(cnt)
    return tot[0] * jnp.float32(1.0 / BATCH_N)


# speedup vs baseline: 3.6270x; 1.0567x over previous
import functools

import jax
import jax.numpy as jnp
from jax import lax
from jax.experimental import pallas as pl
from jax.experimental.pallas import tpu as pltpu
from jax.experimental.pallas import tpu_sc as plsc

TOPK = 5
BATCH_N = 128
VOCAB_N = 100000
LANES = 16
NCORES = 2
NSUB = 16
NLAB = 20
BHALF = BATCH_N // NCORES
NBG = BHALF // LANES
NQ = BATCH_N // LANES
VSLAB = VOCAB_N // NSUB
CH_ROWS = 250
NCHUNK = VSLAB // CH_ROWS
UNROLL = 25
INNER = CH_ROWS // UNROLL
LROWS = 2
NGTILES = NLAB // LROWS

_params = pltpu.CompilerParams(
    needs_layout_passes=False, use_tc_tiling_on_sc=False)
_mesh = plsc.VectorSubcoreMesh(core_axis_name="c", subcore_axis_name="s")


@functools.partial(
    pl.kernel,
    out_type=jax.ShapeDtypeStruct((NCORES, LANES), jnp.float32),
    mesh=_mesh,
    compiler_params=_params,
    scratch_types=[
        pltpu.VMEM((2, CH_ROWS, BHALF), jnp.float32),
        pltpu.VMEM((NLAB, BATCH_N), jnp.int32),
        pltpu.VMEM((LROWS, BATCH_N), jnp.int32),
        pltpu.VMEM((LROWS, BATCH_N, BATCH_N), jnp.float32),
        pltpu.VMEM((BHALF,), jnp.float32),
        pltpu.VMEM((NSUB, BHALF), jnp.float32),
        pltpu.VMEM((NBG, LANES), jnp.float32),
        pltpu.VMEM((BHALF,), jnp.int32),
        pltpu.VMEM((NSUB, BHALF), jnp.int32),
        pltpu.VMEM((LANES,), jnp.float32),
        pltpu.VMEM_SHARED((NSUB, BHALF), jnp.float32),
        pltpu.VMEM_SHARED((NSUB, BHALF), jnp.int32),
        pltpu.SemaphoreType.DMA,
        pltpu.SemaphoreType.DMA,
        pltpu.SemaphoreType.DMA,
    ],
)
def _sc_topk_acc(pt_hbm, labt_hbm, out_hbm,
                 buf, labv, idxg, grow, plm_v, slm_v, lmax_v,
                 cnt_v, sums_v, outv, shared_l, shared_c,
                 sem0, sem1, gsem):
    core = lax.axis_index("c")
    sid = lax.axis_index("s")
    sems = (sem0, sem1)
    iota = lax.iota(jnp.int32, LANES)

    row0 = sid * VSLAB
    col0 = core * BHALF
    q0 = core * NBG

    def chunk_cp(c):
        return pltpu.make_async_copy(
            pt_hbm.at[pl.ds(row0 + c * CH_ROWS, CH_ROWS),
                      pl.ds(col0, BHALF)],
            buf.at[c & 1],
            sems[c & 1],
        )

    chunk_cp(0).start()
    chunk_cp(1).start()

    pltpu.sync_copy(labt_hbm, labv)
    neg = jnp.full((LANES,), -jnp.inf, jnp.float32)
    j0 = sid * LROWS
    gactive = sid < NGTILES

    @pl.when(gactive)
    def _():
        for t in range(LROWS):
            for blk in range(NQ):
                idxg[t, pl.ds(blk * LANES, LANES)] = (
                    labv[j0 + t, pl.ds(blk * LANES, LANES)])
        gathers = [
            pltpu.make_async_copy(pt_hbm.at[idxg.at[t]], grow.at[t], gsem)
            for t in range(LROWS)
        ]
        for g in gathers:
            g.start()
        for g in gathers:
            g.wait()
        for bg in range(NBG):
            m = neg
            bv = col0 + bg * LANES + iota
            for t in range(LROWS):
                tv = jnp.broadcast_to(t, (LANES,)).astype(jnp.int32)
                vals = plsc.load_gather(grow, [tv, bv, bv])
                m = jnp.maximum(m, vals)
            plm_v[pl.ds(bg * LANES, LANES)] = m

    @pl.when(jnp.logical_not(gactive))
    def _():
        for bg in range(NBG):
            plm_v[pl.ds(bg * LANES, LANES)] = neg

    pltpu.sync_copy(plm_v, shared_l.at[sid])
    plsc.subcore_barrier()
    pltpu.sync_copy(shared_l, slm_v)
    for bg in range(NBG):
        m = slm_v[0, pl.ds(bg * LANES, LANES)]
        for r in range(1, NSUB):
            m = jnp.maximum(m, slm_v[r, pl.ds(bg * LANES, LANES)])
        lmax_v[bg] = m

    zero = jnp.zeros((LANES,), jnp.int32)
    ones = jnp.ones((LANES,), jnp.int32)
    for bg in range(NBG):
        cnt_v[pl.ds(bg * LANES, LANES)] = zero

    for c in range(NCHUNK):
        chunk_cp(c).wait()
        if c + 2 < NCHUNK:
            chunk_cp(c + 2).start()
        cb = c & 1

        def bg_body(bg, _, cb=cb):
            lmax = lmax_v[bg]
            boff = bg * LANES

            def row_body(i, accs, cb=cb, boff=boff, lmax=lmax):
                out = []
                for k in range(UNROLL):
                    v = buf[cb, i * UNROLL + k, pl.ds(boff, LANES)]
                    out.append(accs[k] + jnp.where(v > lmax, ones, zero))
                return tuple(out)

            accs = list(lax.fori_loop(0, INNER, row_body, (zero,) * UNROLL))
            while len(accs) > 1:
                nxt = [accs[i] + accs[i + 1]
                       for i in range(0, len(accs) - 1, 2)]
                if len(accs) % 2:
                    nxt.append(accs[-1])
                accs = nxt
            cnt_v[pl.ds(boff, LANES)] = cnt_v[pl.ds(boff, LANES)] + accs[0]
            return 0

        lax.fori_loop(0, NBG, bg_body, 0)

    pltpu.sync_copy(cnt_v, shared_c.at[sid])
    plsc.subcore_barrier()

    @pl.when(sid == 0)
    def _():
        pltpu.sync_copy(shared_c, sums_v)
        nmatch = jnp.float32(0.0)
        for bg in range(NBG):
            tot = sums_v[0, pl.ds(bg * LANES, LANES)]
            for r in range(1, NSUB):
                tot = tot + sums_v[r, pl.ds(bg * LANES, LANES)]
            nmatch = nmatch + jnp.sum(
                jnp.where(tot < TOPK, jnp.float32(1.0), jnp.float32(0.0)))
        outv[...] = jnp.broadcast_to(nmatch, (LANES,))
        pltpu.sync_copy(outv, out_hbm.at[core])


def kernel(probas, labels):
    pt = probas.T
    labt = labels.T
    out = _sc_topk_acc(pt, labt)
    return (out[0, 0] + out[1, 0]) * jnp.float32(1.0 / BATCH_N)
